# Initial kernel scaffold; baseline (speedup 1.0000x reference)
import jax
import jax.numpy as jnp
from jax.experimental import pallas as pl


def kernel(input_ids, attention_mask, token_type_ids, word_embeddings, soft_prompt, W, b):
    raise NotImplementedError("write your pallas kernel here")


# speedup vs baseline: 2.0032x; 2.0032x over previous
import functools

import jax
import jax.numpy as jnp
from jax import lax
from jax.experimental import pallas as pl
from jax.experimental.pallas import tpu as pltpu
from jax.experimental.pallas import tpu_sc as plsc

NC = 2
NS = 16
NW = NC * NS


def _sc_gather_build(tokens, V, D):
    tpw = tokens // NW
    ck = 64
    nchunks = tpw // ck
    mesh = plsc.VectorSubcoreMesh(core_axis_name="c", subcore_axis_name="s")

    @functools.partial(
        pl.kernel,
        mesh=mesh,
        out_type=jax.ShapeDtypeStruct((tokens, D), jnp.float32),
        scratch_types=[
            pltpu.VMEM((ck,), jnp.int32),
            pltpu.VMEM((ck, D), jnp.float32),
            pltpu.SemaphoreType.DMA,
        ],
    )
    def sc_gather(ids_hbm, table_hbm, e_hbm, idx_v, rows_v, sem):
        wid = lax.axis_index("s") * NC + lax.axis_index("c")
        base = wid * tpw
        for k in range(nchunks):
            pltpu.sync_copy(ids_hbm.at[pl.ds(base + k * ck, ck)], idx_v)
            pltpu.async_copy(table_hbm.at[idx_v], rows_v, sem).wait()
            pltpu.sync_copy(rows_v, e_hbm.at[pl.ds(base + k * ck, ck)])

    return sc_gather


def _tok_body(x_ref, w_ref, b_ref, m_ref, o_ref):
    acc = jnp.dot(x_ref[...], w_ref[...], preferred_element_type=jnp.float32)
    o_ref[...] = jnp.tanh(acc + b_ref[...]) * m_ref[...]


def _tok_matmul(x, W, b2, m2):
    rows, D = x.shape
    tile = 512
    grid = rows // tile
    return pl.pallas_call(
        _tok_body,
        grid=(grid,),
        in_specs=[
            pl.BlockSpec((tile, D), lambda i: (i, 0)),
            pl.BlockSpec((D, D), lambda i: (0, 0)),
            pl.BlockSpec((1, D), lambda i: (0, 0)),
            pl.BlockSpec((tile, 1), lambda i: (i, 0)),
        ],
        out_specs=pl.BlockSpec((tile, D), lambda i: (i, 0)),
        out_shape=jax.ShapeDtypeStruct((rows, D), jnp.float32),
        compiler_params=pltpu.CompilerParams(
            dimension_semantics=("parallel",),
        ),
    )(x, W, b2, m2)


def _prompt_body(x_ref, w_ref, b_ref, o_ref):
    acc = jnp.dot(x_ref[...], w_ref[...], preferred_element_type=jnp.float32)
    o_ref[...] = jnp.tanh(acc + b_ref[...])


def _prompt_matmul(sp_pad, W, b2):
    rows, D = sp_pad.shape
    return pl.pallas_call(
        _prompt_body,
        out_shape=jax.ShapeDtypeStruct((rows, D), jnp.float32),
    )(sp_pad, W, b2)


def kernel(input_ids, attention_mask, token_type_ids, word_embeddings,
           soft_prompt, W, b):
    B, S = input_ids.shape
    V, D = word_embeddings.shape
    P = soft_prompt.shape[0]

    ids = input_ids.reshape(-1).astype(jnp.int32)
    sc_gather = _sc_gather_build(B * S, V, D)
    e = sc_gather(ids, word_embeddings)

    b2 = b.reshape(1, D)
    m2 = attention_mask.reshape(-1, 1).astype(jnp.float32)
    h_tok = _tok_matmul(e, W, b2, m2).reshape(B, S, D)

    p_pad = (-P) % 8
    sp_pad = jnp.pad(soft_prompt, ((0, p_pad), (0, 0)))
    h_p = _prompt_matmul(sp_pad, W, b2)[:P]
    h_p = jnp.broadcast_to(h_p[None], (B, P, D))

    return jnp.concatenate([h_p, h_tok], axis=1)


# speedup vs baseline: 2.0085x; 1.0026x over previous
import functools

import jax
import jax.numpy as jnp
from jax import lax
from jax.experimental import pallas as pl
from jax.experimental.pallas import tpu as pltpu
from jax.experimental.pallas import tpu_sc as plsc

NC = 2
NS = 16
NW = NC * NS


def _sc_gather_build(tokens, V, D):
    tpw = tokens // NW
    ck = 64
    nchunks = tpw // ck
    mesh = plsc.VectorSubcoreMesh(core_axis_name="c", subcore_axis_name="s")

    @functools.partial(
        pl.kernel,
        mesh=mesh,
        out_type=jax.ShapeDtypeStruct((tokens, D), jnp.float32),
        scratch_types=[
            pltpu.VMEM((ck,), jnp.int32),
            pltpu.VMEM((ck, D), jnp.float32),
            pltpu.SemaphoreType.DMA,
        ],
        compiler_params=pltpu.CompilerParams(use_tc_tiling_on_sc=True),
    )
    def sc_gather(ids_hbm, table_hbm, e_hbm, idx_v, rows_v, sem):
        wid = lax.axis_index("s") * NC + lax.axis_index("c")
        base = wid * tpw
        for k in range(nchunks):
            pltpu.sync_copy(ids_hbm.at[pl.ds(base + k * ck, ck)], idx_v)
            pltpu.async_copy(table_hbm.at[idx_v], rows_v, sem).wait()
            pltpu.sync_copy(rows_v, e_hbm.at[pl.ds(base + k * ck, ck)])

    return sc_gather


def _tok_body(x_ref, w_ref, b_ref, m_ref, o_ref):
    x = x_ref[...].astype(jnp.bfloat16)
    w = w_ref[...].astype(jnp.bfloat16)
    acc = jnp.dot(x, w, preferred_element_type=jnp.float32)
    o_ref[...] = jnp.tanh(acc + b_ref[...]) * m_ref[...]


def _tok_matmul(x, W, b2, m2):
    rows, D = x.shape
    tile = 512
    grid = rows // tile
    return pl.pallas_call(
        _tok_body,
        grid=(grid,),
        in_specs=[
            pl.BlockSpec((tile, D), lambda i: (i, 0)),
            pl.BlockSpec((D, D), lambda i: (0, 0)),
            pl.BlockSpec((1, D), lambda i: (0, 0)),
            pl.BlockSpec((tile, 1), lambda i: (i, 0)),
        ],
        out_specs=pl.BlockSpec((tile, D), lambda i: (i, 0)),
        out_shape=jax.ShapeDtypeStruct((rows, D), jnp.float32),
        compiler_params=pltpu.CompilerParams(
            dimension_semantics=("parallel",),
        ),
    )(x, W, b2, m2)


def _prompt_body(x_ref, w_ref, b_ref, o_ref):
    acc = jnp.dot(x_ref[...], w_ref[...], preferred_element_type=jnp.float32)
    o_ref[...] = jnp.tanh(acc + b_ref[...])


def _prompt_matmul(sp_pad, W, b2):
    rows, D = sp_pad.shape
    return pl.pallas_call(
        _prompt_body,
        out_shape=jax.ShapeDtypeStruct((rows, D), jnp.float32),
    )(sp_pad, W, b2)


def kernel(input_ids, attention_mask, token_type_ids, word_embeddings,
           soft_prompt, W, b):
    B, S = input_ids.shape
    V, D = word_embeddings.shape
    P = soft_prompt.shape[0]

    ids = input_ids.reshape(-1).astype(jnp.int32)
    sc_gather = _sc_gather_build(B * S, V, D)
    e = sc_gather(ids, word_embeddings)

    b2 = b.reshape(1, D)
    m2 = attention_mask.reshape(-1, 1).astype(jnp.float32)
    h_tok = _tok_matmul(e, W, b2, m2).reshape(B, S, D)

    p_pad = (-P) % 8
    sp_pad = jnp.pad(soft_prompt, ((0, p_pad), (0, 0)))
    h_p = _prompt_matmul(sp_pad, W, b2)[:P]
    h_p = jnp.broadcast_to(h_p[None], (B, P, D))

    return jnp.concatenate([h_p, h_tok], axis=1)
